# Initial kernel scaffold; baseline (speedup 1.0000x reference)
import jax
import jax.numpy as jnp
from jax.experimental import pallas as pl


def kernel(user_id, f_age, f_city, f_device, f_hist_item, dense, W_user, W_age, W_city, W_device, W_hist):
    raise NotImplementedError("write your pallas kernel here")


# speedup vs baseline: 1.3775x; 1.3775x over previous
import jax
import jax.numpy as jnp
from jax import lax
from jax.experimental import pallas as pl
from jax.experimental.pallas import tpu as pltpu
from jax.experimental.pallas import tpu_sc as plsc

B = 16384
L = 50
D_MAIN = 32
D_SPARSE = 16
N_DENSE = 8
D_OUT = 2 * D_MAIN + 3 * D_SPARSE + N_DENSE

NC = 2
NS = 16
NW = NC * NS
BW = B // NW
NG = BW // 16


def _body(emT, eaT, ecT, edT, ehT, hT, dT, outT,
          hist_v, slab0, slab1, acc, sbuf0, sbuf1,
          sem_h, sem_s0, sem_s1, sem_o0, sem_o1):
    wid = lax.axis_index("s") * NC + lax.axis_index("c")
    base = pl.multiple_of(wid * BW, BW)

    c_hv = pltpu.async_copy(hT.at[:, pl.ds(base, BW)], hist_v, sem_h)

    slabs = (slab0, slab1)
    sems = (sem_s0, sem_s1)

    def issue(l, p):
        off = pl.multiple_of(l * B + base, 128)
        pltpu.async_copy(ehT.at[:, pl.ds(off, BW)], slabs[p], sems[p])

    def drain(p):
        pltpu.make_async_copy(ehT.at[:, pl.ds(0, BW)], slabs[p], sems[p]).wait()

    issue(0, 0)
    issue(1, 1)

    zv = jnp.zeros((16,), jnp.float32)

    def zero_g(g, carry):
        for d in range(D_MAIN):
            acc[d, pl.ds(g * 16, 16)] = zv
        return carry

    lax.fori_loop(0, NG, zero_g, 0)

    def accum(l, p):
        drain(p)
        slab = slabs[p]

        def add_g(g, carry):
            for d in range(D_MAIN):
                plsc.addupdate(acc.at[d, pl.ds(g * 16, 16)],
                               slab[d, pl.ds(g * 16, 16)])
            return carry

        lax.fori_loop(0, NG, add_g, 0)

    def hist_loop(lp, carry):
        l = lp * 2
        accum(l, 0)

        @pl.when(l + 2 < L)
        def _():
            issue(l + 2, 0)

        accum(l + 1, 1)

        @pl.when(l + 3 < L)
        def _():
            issue(l + 3, 1)

        return carry

    lax.fori_loop(0, L // 2, hist_loop, 0)

    c_hv.wait()
    one = jnp.ones((16,), jnp.float32)
    zero = jnp.zeros((16,), jnp.float32)

    def scale_g(g, carry):
        cnt = jnp.zeros((16,), jnp.float32)
        for l in range(L):
            v = hist_v[l, pl.ds(g * 16, 16)]
            cnt = cnt + jnp.where(v != 0, one, zero)
        inv = 1.0 / jnp.maximum(cnt, 1.0)
        for d in range(D_MAIN):
            acc[d, pl.ds(g * 16, 16)] = acc[d, pl.ds(g * 16, 16)] * inv
        return carry

    lax.fori_loop(0, NG, scale_g, 0)

    pooled_out = pltpu.async_copy(
        acc, outT.at[pl.ds(80, D_MAIN), pl.ds(base, BW)], sem_s1)

    pieces = [
        (emT, 0, 0, 16),
        (emT, 16, 16, 16),
        (eaT, 0, 32, 16),
        (ecT, 0, 48, 16),
        (edT, 0, 64, 16),
        (dT, 0, 112, 8),
    ]
    bufs = (sbuf0, sbuf1)
    isems = (sem_h, sem_s0)
    osems = (sem_o0, sem_o1)

    def issue_in(i):
        src, srow, orow, n = pieces[i]
        bb = bufs[i % 2].at[pl.ds(0, n)]
        return (pltpu.async_copy(
            src.at[pl.ds(srow, n), pl.ds(base, BW)], bb, isems[i % 2]), bb, orow, n)

    pending_in = [issue_in(0), issue_in(1)]
    pending_out = []
    for i in range(len(pieces)):
        c, bb, orow, n = pending_in[i]
        c.wait()
        oc = pltpu.async_copy(
            bb, outT.at[pl.ds(orow, n), pl.ds(base, BW)], osems[i % 2])
        pending_out.append(oc)
        if i + 2 < len(pieces):
            oc.wait()
            pending_out.pop()
            pending_in.append(issue_in(i + 2))
    for c in pending_out:
        c.wait()
    pooled_out.wait()


@jax.jit
def kernel(user_id, f_age, f_city, f_device, f_hist_item, dense,
           W_user, W_age, W_city, W_device, W_hist):
    idxT = f_hist_item.T.reshape(-1)
    e_hist = jnp.take(W_hist, idxT, axis=0, mode="clip")
    e_main = jnp.take(W_user, user_id, axis=0, mode="clip")
    e_age = jnp.take(W_age, f_age, axis=0, mode="clip")
    e_city = jnp.take(W_city, f_city, axis=0, mode="clip")
    e_dev = jnp.take(W_device, f_device, axis=0, mode="clip")

    mesh = plsc.VectorSubcoreMesh(core_axis_name="c", subcore_axis_name="s")
    run = pl.kernel(
        _body,
        out_type=jax.ShapeDtypeStruct((D_OUT, B), jnp.float32),
        mesh=mesh,
        scratch_types=[
            pltpu.VMEM((L, BW), jnp.int32),
            pltpu.VMEM((D_MAIN, BW), jnp.float32),
            pltpu.VMEM((D_MAIN, BW), jnp.float32),
            pltpu.VMEM((D_MAIN, BW), jnp.float32),
            pltpu.VMEM((16, BW), jnp.float32),
            pltpu.VMEM((16, BW), jnp.float32),
            pltpu.SemaphoreType.DMA,
            pltpu.SemaphoreType.DMA,
            pltpu.SemaphoreType.DMA,
            pltpu.SemaphoreType.DMA,
            pltpu.SemaphoreType.DMA,
        ],
    )
    outT = run(e_main.T, e_age.T, e_city.T, e_dev.T, e_hist.T,
               f_hist_item.T, dense.T)
    return outT.T


# speedup vs baseline: 1.4311x; 1.0389x over previous
import jax
import jax.numpy as jnp
from jax import lax
from jax.experimental import pallas as pl
from jax.experimental.pallas import tpu as pltpu
from jax.experimental.pallas import tpu_sc as plsc

B = 16384
L = 50
D_MAIN = 32
D_SPARSE = 16
N_DENSE = 8
D_OUT = 2 * D_MAIN + 3 * D_SPARSE + N_DENSE

NC = 2
NS = 16
NW = NC * NS
BW = B // NW
NG = BW // 16
NPAIR = L // 2


def _body(emT, eaT, ecT, edT, ehT, hT, dT, outT,
          hist_v, b0, b1, b2, b3, acc,
          m0, m1, m2, m3, sem_h):
    wid = lax.axis_index("s") * NC + lax.axis_index("c")
    base = pl.multiple_of(wid * BW, BW)
    bufs = (b0, b1, b2, b3)
    sems = (m0, m1, m2, m3)

    c_hv = pltpu.async_copy(hT.at[:, pl.ds(base, BW)], hist_v, sem_h)

    def issue(l, bi):
        off = pl.multiple_of(l * B + base, 128)
        pltpu.async_copy(ehT.at[:, pl.ds(off, BW)], bufs[bi], sems[bi])

    def issue_when(l, bi):
        @pl.when(l < L)
        def _():
            issue(l, bi)

    def drain(bi):
        pltpu.make_async_copy(ehT.at[:, pl.ds(0, BW)], bufs[bi], sems[bi]).wait()

    def accum(bx, by, first):
        drain(bx)
        drain(by)
        X, Y = bufs[bx], bufs[by]

        def add_g(g, carry):
            sl = pl.ds(g * 16, 16)
            for d in range(D_MAIN):
                v = X[d, sl] + Y[d, sl]
                if first:
                    acc[d, sl] = v
                else:
                    plsc.addupdate(acc.at[d, sl], v)
            return carry

        lax.fori_loop(0, NG, add_g, 0)

    for l in range(4):
        issue(l, l)

    accum(0, 1, True)
    issue(4, 0)
    issue(5, 1)

    def hist_loop(q, carry):
        accum(2, 3, False)
        issue_when(4 * q + 6, 2)
        issue_when(4 * q + 7, 3)
        accum(0, 1, False)
        issue_when(4 * q + 8, 0)
        issue_when(4 * q + 9, 1)
        return carry

    lax.fori_loop(0, (NPAIR - 1) // 2, hist_loop, 0)

    bsl = pl.ds(base, BW)
    ins = [
        [pltpu.async_copy(emT.at[:, bsl], b0, m0)],
        [pltpu.async_copy(eaT.at[:, bsl], b1.at[pl.ds(0, 16)], m1),
         pltpu.async_copy(ecT.at[:, bsl], b1.at[pl.ds(16, 16)], m1)],
        [pltpu.async_copy(edT.at[:, bsl], b2.at[pl.ds(0, 16)], m2),
         pltpu.async_copy(dT.at[:, bsl], b2.at[pl.ds(16, 8)], m2)],
    ]

    c_hv.wait()
    one = jnp.ones((16,), jnp.float32)
    zero = jnp.zeros((16,), jnp.float32)

    def scale_g(g, carry):
        sl = pl.ds(g * 16, 16)
        cnt = jnp.zeros((16,), jnp.float32)
        for l in range(L):
            v = hist_v[l, sl]
            cnt = cnt + jnp.where(v != 0, one, zero)
        inv = 1.0 / jnp.maximum(cnt, 1.0)
        for d in range(D_MAIN):
            acc[d, sl] = acc[d, sl] * inv
        return carry

    lax.fori_loop(0, NG, scale_g, 0)

    pooled_out = pltpu.async_copy(
        acc, outT.at[pl.ds(80, D_MAIN), pl.ds(base, BW)], sem_h)

    outs = [
        [(b0, 0, 32)],
        [(b1.at[pl.ds(0, 16)], 32, 16), (b1.at[pl.ds(16, 16)], 48, 16)],
        [(b2.at[pl.ds(0, 16)], 64, 16), (b2.at[pl.ds(16, 8)], 112, 8)],
    ]
    pending = []
    for i in range(3):
        for c in ins[i]:
            c.wait()
        for src, orow, n in outs[i]:
            pending.append(pltpu.async_copy(
                src, outT.at[pl.ds(orow, n), pl.ds(base, BW)], sems[i]))
    for c in pending:
        c.wait()
    pooled_out.wait()


@jax.jit
def kernel(user_id, f_age, f_city, f_device, f_hist_item, dense,
           W_user, W_age, W_city, W_device, W_hist):
    idxT = f_hist_item.T.reshape(-1)
    e_hist = jnp.take(W_hist, idxT, axis=0, mode="clip")
    e_main = jnp.take(W_user, user_id, axis=0, mode="clip")
    e_age = jnp.take(W_age, f_age, axis=0, mode="clip")
    e_city = jnp.take(W_city, f_city, axis=0, mode="clip")
    e_dev = jnp.take(W_device, f_device, axis=0, mode="clip")

    mesh = plsc.VectorSubcoreMesh(core_axis_name="c", subcore_axis_name="s")
    run = pl.kernel(
        _body,
        out_type=jax.ShapeDtypeStruct((D_OUT, B), jnp.float32),
        mesh=mesh,
        scratch_types=[
            pltpu.VMEM((L, BW), jnp.int32),
            pltpu.VMEM((D_MAIN, BW), jnp.float32),
            pltpu.VMEM((D_MAIN, BW), jnp.float32),
            pltpu.VMEM((D_MAIN, BW), jnp.float32),
            pltpu.VMEM((D_MAIN, BW), jnp.float32),
            pltpu.VMEM((D_MAIN, BW), jnp.float32),
            pltpu.SemaphoreType.DMA,
            pltpu.SemaphoreType.DMA,
            pltpu.SemaphoreType.DMA,
            pltpu.SemaphoreType.DMA,
            pltpu.SemaphoreType.DMA,
        ],
    )
    outT = run(e_main.T, e_age.T, e_city.T, e_dev.T, e_hist.T,
               f_hist_item.T, dense.T)
    return outT.T


# speedup vs baseline: 1.5317x; 1.0703x over previous
import jax
import jax.numpy as jnp
from jax import lax
from jax.experimental import pallas as pl
from jax.experimental.pallas import tpu as pltpu
from jax.experimental.pallas import tpu_sc as plsc

B = 16384
L = 50
D_MAIN = 32
D_SPARSE = 16
N_DENSE = 8
D_OUT = 2 * D_MAIN + 3 * D_SPARSE + N_DENSE

NC = 2
NS = 16
NW = NC * NS
BW = B // NW
NG = BW // 16
NPAIR = L // 2


def _body(emT, eaT, ecT, edT, ehT, hT, dT, outT,
          hist_v, b0, b1, b2, b3, acc, invb,
          m0, m1, m2, m3, sem_h):
    wid = lax.axis_index("s") * NC + lax.axis_index("c")
    base = pl.multiple_of(wid * BW, BW)
    bufs = (b0, b1, b2, b3)
    sems = (m0, m1, m2, m3)

    c_hv = pltpu.async_copy(hT.at[:, pl.ds(base, BW)], hist_v, sem_h)

    def issue(l, bi):
        off = pl.multiple_of(l * B + base, 128)
        pltpu.async_copy(ehT.at[:, pl.ds(off, BW)], bufs[bi], sems[bi])

    def issue_when(l, bi):
        @pl.when(l < L)
        def _():
            issue(l, bi)

    def drain(bi):
        pltpu.make_async_copy(ehT.at[:, pl.ds(0, BW)], bufs[bi], sems[bi]).wait()

    def accum(bx, by, first):
        drain(bx)
        drain(by)
        X, Y = bufs[bx], bufs[by]

        def add_one(g):
            sl = pl.ds(g * 16, 16)
            for d0 in range(0, D_MAIN, 8):
                xs = [X[d, sl] for d in range(d0, d0 + 8)]
                ys = [Y[d, sl] for d in range(d0, d0 + 8)]
                for i, d in enumerate(range(d0, d0 + 8)):
                    v = xs[i] + ys[i]
                    if first:
                        acc[d, sl] = v
                    else:
                        plsc.addupdate(acc.at[d, sl], v)

        def add_g(g, carry):
            add_one(g * 2)
            add_one(g * 2 + 1)
            return carry

        lax.fori_loop(0, NG // 2, add_g, 0)

    for l in range(4):
        issue(l, l)

    c_hv.wait()
    one = jnp.ones((16,), jnp.float32)
    zero = jnp.zeros((16,), jnp.float32)

    def count_g(g, carry):
        sl = pl.ds(g * 16, 16)
        cnt = jnp.zeros((16,), jnp.float32)
        for l in range(L):
            v = hist_v[l, sl]
            cnt = cnt + jnp.where(v != 0, one, zero)
        invb[sl] = 1.0 / jnp.maximum(cnt, 1.0)
        return carry

    lax.fori_loop(0, NG, count_g, 0)

    accum(0, 1, True)
    issue(4, 0)
    issue(5, 1)

    def hist_loop(q, carry):
        accum(2, 3, False)
        issue_when(4 * q + 6, 2)
        issue_when(4 * q + 7, 3)
        accum(0, 1, False)
        issue_when(4 * q + 8, 0)
        issue_when(4 * q + 9, 1)
        return carry

    lax.fori_loop(0, (NPAIR - 1) // 2, hist_loop, 0)

    bsl = pl.ds(base, BW)
    ins = [
        [pltpu.async_copy(emT.at[:, bsl], b0, m0)],
        [pltpu.async_copy(eaT.at[:, bsl], b1.at[pl.ds(0, 16)], m1),
         pltpu.async_copy(ecT.at[:, bsl], b1.at[pl.ds(16, 16)], m1)],
        [pltpu.async_copy(edT.at[:, bsl], b2.at[pl.ds(0, 16)], m2),
         pltpu.async_copy(dT.at[:, bsl], b2.at[pl.ds(16, 8)], m2)],
    ]

    def scale_g(g, carry):
        sl = pl.ds(g * 16, 16)
        inv = invb[sl]
        for d in range(D_MAIN):
            acc[d, sl] = acc[d, sl] * inv
        return carry

    lax.fori_loop(0, NG, scale_g, 0)

    pooled_out = pltpu.async_copy(
        acc, outT.at[pl.ds(80, D_MAIN), pl.ds(base, BW)], sem_h)

    outs = [
        [(b0, 0, 32)],
        [(b1.at[pl.ds(0, 16)], 32, 16), (b1.at[pl.ds(16, 16)], 48, 16)],
        [(b2.at[pl.ds(0, 16)], 64, 16), (b2.at[pl.ds(16, 8)], 112, 8)],
    ]
    pending = []
    for i in range(3):
        for c in ins[i]:
            c.wait()
        for src, orow, n in outs[i]:
            pending.append(pltpu.async_copy(
                src, outT.at[pl.ds(orow, n), pl.ds(base, BW)], sems[i]))
    for c in pending:
        c.wait()
    pooled_out.wait()


@jax.jit
def kernel(user_id, f_age, f_city, f_device, f_hist_item, dense,
           W_user, W_age, W_city, W_device, W_hist):
    idxT = f_hist_item.T.reshape(-1)
    mode = "promise_in_bounds"
    e_hist = W_hist.at[idxT].get(mode=mode)
    e_main = W_user.at[user_id].get(mode=mode)
    e_age = W_age.at[f_age].get(mode=mode)
    e_city = W_city.at[f_city].get(mode=mode)
    e_dev = W_device.at[f_device].get(mode=mode)

    mesh = plsc.VectorSubcoreMesh(core_axis_name="c", subcore_axis_name="s")
    run = pl.kernel(
        _body,
        out_type=jax.ShapeDtypeStruct((D_OUT, B), jnp.float32),
        mesh=mesh,
        scratch_types=[
            pltpu.VMEM((L, BW), jnp.int32),
            pltpu.VMEM((D_MAIN, BW), jnp.float32),
            pltpu.VMEM((D_MAIN, BW), jnp.float32),
            pltpu.VMEM((D_MAIN, BW), jnp.float32),
            pltpu.VMEM((D_MAIN, BW), jnp.float32),
            pltpu.VMEM((D_MAIN, BW), jnp.float32),
            pltpu.VMEM((BW,), jnp.float32),
            pltpu.SemaphoreType.DMA,
            pltpu.SemaphoreType.DMA,
            pltpu.SemaphoreType.DMA,
            pltpu.SemaphoreType.DMA,
            pltpu.SemaphoreType.DMA,
        ],
    )
    outT = run(e_main.T, e_age.T, e_city.T, e_dev.T, e_hist.T,
               f_hist_item.T, dense.T)
    return outT.T


# speedup vs baseline: 1.5319x; 1.0001x over previous
import jax
import jax.numpy as jnp
from jax import lax
from jax.experimental import pallas as pl
from jax.experimental.pallas import tpu as pltpu
from jax.experimental.pallas import tpu_sc as plsc

B = 16384
L = 50
D_MAIN = 32
D_SPARSE = 16
N_DENSE = 8
D_OUT = 2 * D_MAIN + 3 * D_SPARSE + N_DENSE

NC = 2
NS = 16
NW = NC * NS
BW = B // NW
NG = BW // 16
NPAIR = L // 2


def _body(emT, eaT, ecT, edT, ehT, hT, dT, outT,
          hist_v, b0, b1, b2, b3, acc, invb,
          m0, m1, m2, m3, sem_h):
    wid = lax.axis_index("s") * NC + lax.axis_index("c")
    base = pl.multiple_of(wid * BW, BW)
    bufs = (b0, b1, b2, b3)
    sems = (m0, m1, m2, m3)

    c_hv = pltpu.async_copy(hT.at[:, pl.ds(base, BW)], hist_v, sem_h)

    def issue(l, bi):
        off = pl.multiple_of(l * B + base, 128)
        pltpu.async_copy(ehT.at[:, pl.ds(off, BW)], bufs[bi], sems[bi])

    def issue_when(l, bi):
        @pl.when(l < L)
        def _():
            issue(l, bi)

    def drain(bi):
        pltpu.make_async_copy(ehT.at[:, pl.ds(0, BW)], bufs[bi], sems[bi]).wait()

    def accum(bx, by, first):
        drain(bx)
        drain(by)
        X, Y = bufs[bx], bufs[by]

        def add_one(g):
            sl = pl.ds(g * 16, 16)
            for d0 in range(0, D_MAIN, 8):
                xs = [X[d, sl] for d in range(d0, d0 + 8)]
                ys = [Y[d, sl] for d in range(d0, d0 + 8)]
                for i, d in enumerate(range(d0, d0 + 8)):
                    v = xs[i] + ys[i]
                    if first:
                        acc[d, sl] = v
                    else:
                        plsc.addupdate(acc.at[d, sl], v)

        def add_g(g, carry):
            add_one(g * 2)
            add_one(g * 2 + 1)
            return carry

        lax.fori_loop(0, NG // 2, add_g, 0)

    for l in range(4):
        issue(l, l)

    c_hv.wait()
    one = jnp.ones((16,), jnp.float32)
    zero = jnp.zeros((16,), jnp.float32)

    def count_g(g, carry):
        sl = pl.ds(g * 16, 16)
        cnt = jnp.zeros((16,), jnp.float32)
        for l in range(L):
            v = hist_v[l, sl]
            cnt = cnt + jnp.where(v != 0, one, zero)
        invb[sl] = 1.0 / jnp.maximum(cnt, 1.0)
        return carry

    lax.fori_loop(0, NG, count_g, 0)

    accum(0, 1, True)
    issue(4, 0)
    issue(5, 1)

    def hist_loop(q, carry):
        accum(2, 3, False)
        issue_when(4 * q + 6, 2)
        issue_when(4 * q + 7, 3)
        accum(0, 1, False)
        issue_when(4 * q + 8, 0)
        issue_when(4 * q + 9, 1)
        return carry

    lax.fori_loop(0, (NPAIR - 1) // 2, hist_loop, 0)

    bsl = pl.ds(base, BW)
    ins = [
        [pltpu.async_copy(emT.at[:, bsl], b0, m0)],
        [pltpu.async_copy(eaT.at[:, bsl], b1.at[pl.ds(0, 16)], m1),
         pltpu.async_copy(ecT.at[:, bsl], b1.at[pl.ds(16, 16)], m1)],
        [pltpu.async_copy(edT.at[:, bsl], b2.at[pl.ds(0, 16)], m2),
         pltpu.async_copy(dT.at[:, bsl], b2.at[pl.ds(16, 8)], m2)],
    ]

    def scale_g(g, carry):
        sl = pl.ds(g * 16, 16)
        inv = invb[sl]
        for d in range(D_MAIN):
            acc[d, sl] = acc[d, sl] * inv
        return carry

    lax.fori_loop(0, NG, scale_g, 0)

    pooled_out = pltpu.async_copy(
        acc, outT.at[pl.ds(80, D_MAIN), pl.ds(base, BW)], sem_h)

    outs = [
        [(b0, 0, 32)],
        [(b1.at[pl.ds(0, 16)], 32, 16), (b1.at[pl.ds(16, 16)], 48, 16)],
        [(b2.at[pl.ds(0, 16)], 64, 16), (b2.at[pl.ds(16, 8)], 112, 8)],
    ]
    pending = []
    for i in range(3):
        for c in ins[i]:
            c.wait()
        for src, orow, n in outs[i]:
            pending.append(pltpu.async_copy(
                src, outT.at[pl.ds(orow, n), pl.ds(base, BW)], sems[i]))
    for c in pending:
        c.wait()
    pooled_out.wait()


@jax.jit
def kernel(user_id, f_age, f_city, f_device, f_hist_item, dense,
           W_user, W_age, W_city, W_device, W_hist):
    idxT = f_hist_item.T.reshape(-1)
    mode = "promise_in_bounds"
    e_hist = W_hist.at[idxT].get(mode=mode)
    e_main = jnp.take(W_user, user_id, axis=0, mode="clip")
    e_age = jnp.take(W_age, f_age, axis=0, mode="clip")
    e_city = jnp.take(W_city, f_city, axis=0, mode="clip")
    e_dev = jnp.take(W_device, f_device, axis=0, mode="clip")

    mesh = plsc.VectorSubcoreMesh(core_axis_name="c", subcore_axis_name="s")
    run = pl.kernel(
        _body,
        out_type=jax.ShapeDtypeStruct((D_OUT, B), jnp.float32),
        mesh=mesh,
        scratch_types=[
            pltpu.VMEM((L, BW), jnp.int32),
            pltpu.VMEM((D_MAIN, BW), jnp.float32),
            pltpu.VMEM((D_MAIN, BW), jnp.float32),
            pltpu.VMEM((D_MAIN, BW), jnp.float32),
            pltpu.VMEM((D_MAIN, BW), jnp.float32),
            pltpu.VMEM((D_MAIN, BW), jnp.float32),
            pltpu.VMEM((BW,), jnp.float32),
            pltpu.SemaphoreType.DMA,
            pltpu.SemaphoreType.DMA,
            pltpu.SemaphoreType.DMA,
            pltpu.SemaphoreType.DMA,
            pltpu.SemaphoreType.DMA,
        ],
    )
    outT = run(e_main.T, e_age.T, e_city.T, e_dev.T, e_hist.T,
               f_hist_item.T, dense.T)
    return outT.T


# speedup vs baseline: 1.5399x; 1.0052x over previous
import jax
import jax.numpy as jnp
from jax import lax
from jax.experimental import pallas as pl
from jax.experimental.pallas import tpu as pltpu
from jax.experimental.pallas import tpu_sc as plsc

B = 16384
L = 50
D_MAIN = 32
D_SPARSE = 16
N_DENSE = 8
D_OUT = 2 * D_MAIN + 3 * D_SPARSE + N_DENSE

NC = 2
NS = 16
NW = NC * NS
BW = B // NW
NG = BW // 16
NPAIR = L // 2


def _body(emT, eaT, ecT, edT, ehT, hT, dT, outT,
          hist_v, b0, b1, b2, b3, acc, invb,
          m0, m1, m2, m3, sem_h):
    wid = lax.axis_index("s") * NC + lax.axis_index("c")
    base = pl.multiple_of(wid * BW, BW)
    bufs = (b0, b1, b2, b3)
    sems = (m0, m1, m2, m3)

    c_hv = pltpu.async_copy(hT.at[:, pl.ds(base, BW)], hist_v, sem_h)

    def issue(l, bi):
        off = pl.multiple_of(l * B + base, 128)
        pltpu.async_copy(ehT.at[:, pl.ds(off, BW)], bufs[bi], sems[bi])

    def issue_when(l, bi):
        @pl.when(l < L)
        def _():
            issue(l, bi)

    def drain(bi):
        pltpu.make_async_copy(ehT.at[:, pl.ds(0, BW)], bufs[bi], sems[bi]).wait()

    def accum(bx, by, first):
        drain(bx)
        drain(by)
        X, Y = bufs[bx], bufs[by]

        def add_one(g):
            sl = pl.ds(g * 16, 16)
            for d0 in range(0, D_MAIN, 8):
                xs = [X[d, sl] for d in range(d0, d0 + 8)]
                ys = [Y[d, sl] for d in range(d0, d0 + 8)]
                for i, d in enumerate(range(d0, d0 + 8)):
                    v = xs[i] + ys[i]
                    if first:
                        acc[d, sl] = v
                    else:
                        plsc.addupdate(acc.at[d, sl], v)

        def add_g(g, carry):
            for k in range(4):
                add_one(g * 4 + k)
            return carry

        lax.fori_loop(0, NG // 4, add_g, 0)

    for l in range(4):
        issue(l, l)

    c_hv.wait()
    one = jnp.ones((16,), jnp.float32)
    zero = jnp.zeros((16,), jnp.float32)

    def count_g(g, carry):
        sl = pl.ds(g * 16, 16)
        cnt = jnp.zeros((16,), jnp.float32)
        for l in range(L):
            v = hist_v[l, sl]
            cnt = cnt + jnp.where(v != 0, one, zero)
        invb[sl] = 1.0 / jnp.maximum(cnt, 1.0)
        return carry

    lax.fori_loop(0, NG, count_g, 0)

    accum(0, 1, True)
    issue(4, 0)
    issue(5, 1)

    def hist_loop(q, carry):
        accum(2, 3, False)
        issue_when(4 * q + 6, 2)
        issue_when(4 * q + 7, 3)
        accum(0, 1, False)
        issue_when(4 * q + 8, 0)
        issue_when(4 * q + 9, 1)
        return carry

    lax.fori_loop(0, (NPAIR - 1) // 2, hist_loop, 0)

    bsl = pl.ds(base, BW)
    ins = [
        [pltpu.async_copy(emT.at[:, bsl], b0, m0)],
        [pltpu.async_copy(eaT.at[:, bsl], b1.at[pl.ds(0, 16)], m1),
         pltpu.async_copy(ecT.at[:, bsl], b1.at[pl.ds(16, 16)], m1)],
        [pltpu.async_copy(edT.at[:, bsl], b2.at[pl.ds(0, 16)], m2),
         pltpu.async_copy(dT.at[:, bsl], b2.at[pl.ds(16, 8)], m2)],
    ]

    def scale_g(g, carry):
        sl = pl.ds(g * 16, 16)
        inv = invb[sl]
        for d in range(D_MAIN):
            acc[d, sl] = acc[d, sl] * inv
        return carry

    lax.fori_loop(0, NG, scale_g, 0)

    pooled_out = pltpu.async_copy(
        acc, outT.at[pl.ds(80, D_MAIN), pl.ds(base, BW)], sem_h)

    outs = [
        [(b0, 0, 32)],
        [(b1.at[pl.ds(0, 16)], 32, 16), (b1.at[pl.ds(16, 16)], 48, 16)],
        [(b2.at[pl.ds(0, 16)], 64, 16), (b2.at[pl.ds(16, 8)], 112, 8)],
    ]
    pending = []
    for i in range(3):
        for c in ins[i]:
            c.wait()
        for src, orow, n in outs[i]:
            pending.append(pltpu.async_copy(
                src, outT.at[pl.ds(orow, n), pl.ds(base, BW)], sems[i]))
    for c in pending:
        c.wait()
    pooled_out.wait()


@jax.jit
def kernel(user_id, f_age, f_city, f_device, f_hist_item, dense,
           W_user, W_age, W_city, W_device, W_hist):
    idxT = f_hist_item.T.reshape(-1)
    mode = "promise_in_bounds"
    e_hist = W_hist.at[idxT].get(mode=mode)
    e_main = jnp.take(W_user, user_id, axis=0, mode="clip")
    e_age = jnp.take(W_age, f_age, axis=0, mode="clip")
    e_city = jnp.take(W_city, f_city, axis=0, mode="clip")
    e_dev = jnp.take(W_device, f_device, axis=0, mode="clip")

    mesh = plsc.VectorSubcoreMesh(core_axis_name="c", subcore_axis_name="s")
    run = pl.kernel(
        _body,
        out_type=jax.ShapeDtypeStruct((D_OUT, B), jnp.float32),
        mesh=mesh,
        scratch_types=[
            pltpu.VMEM((L, BW), jnp.int32),
            pltpu.VMEM((D_MAIN, BW), jnp.float32),
            pltpu.VMEM((D_MAIN, BW), jnp.float32),
            pltpu.VMEM((D_MAIN, BW), jnp.float32),
            pltpu.VMEM((D_MAIN, BW), jnp.float32),
            pltpu.VMEM((D_MAIN, BW), jnp.float32),
            pltpu.VMEM((BW,), jnp.float32),
            pltpu.SemaphoreType.DMA,
            pltpu.SemaphoreType.DMA,
            pltpu.SemaphoreType.DMA,
            pltpu.SemaphoreType.DMA,
            pltpu.SemaphoreType.DMA,
        ],
    )
    outT = run(e_main.T, e_age.T, e_city.T, e_dev.T, e_hist.T,
               f_hist_item.T, dense.T)
    return outT.T
